# Initial kernel scaffold; baseline (speedup 1.0000x reference)
import jax
import jax.numpy as jnp
from jax.experimental import pallas as pl


def kernel(constants_entity, tuples_p1, tuples_p2, W_p1, b_p1, W_p2, b_p2):
    raise NotImplementedError("write your pallas kernel here")


# speedup vs baseline: 3.0036x; 3.0036x over previous
import functools

import jax
import jax.numpy as jnp
from jax import lax
from jax.experimental import pallas as pl
from jax.experimental.pallas import tpu as pltpu
from jax.experimental.pallas import tpu_sc as plsc

_NC = 2
_NS = 16
_NW = _NC * _NS
_LANES = 16


def _prep_body(c_ref, w1_ref, b1_ref, w2_ref, b2_ref, t1a_ref, t1b_ref, t2_ref):
    c = c_ref[...]
    d = c.shape[1]
    t1a_ref[...] = (
        jnp.dot(c, w1_ref[:d, :], preferred_element_type=jnp.float32) + b1_ref[...]
    )
    t1b_ref[...] = jnp.dot(c, w1_ref[d:, :], preferred_element_type=jnp.float32)
    t2_ref[...] = (
        jnp.dot(c, w2_ref[...], preferred_element_type=jnp.float32) + b2_ref[...]
    )


def _make_tables(constants, w1, b1, w2, b2):
    v, d = constants.shape
    a = w1.shape[1]
    bm = 2000
    assert v % bm == 0
    grid = (v // bm,)
    ts = jax.ShapeDtypeStruct((v, a), jnp.float32)
    return pl.pallas_call(
        _prep_body,
        grid=grid,
        in_specs=[
            pl.BlockSpec((bm, d), lambda i: (i, 0)),
            pl.BlockSpec((2 * d, a), lambda i: (0, 0)),
            pl.BlockSpec((1, a), lambda i: (0, 0)),
            pl.BlockSpec((d, a), lambda i: (0, 0)),
            pl.BlockSpec((1, a), lambda i: (0, 0)),
        ],
        out_specs=[
            pl.BlockSpec((bm, a), lambda i: (i, 0)),
            pl.BlockSpec((bm, a), lambda i: (i, 0)),
            pl.BlockSpec((bm, a), lambda i: (i, 0)),
        ],
        out_shape=[ts, ts, ts],
    )(constants, w1, b1.reshape(1, a), w2, b2.reshape(1, a))


def _pick_chunk(n):
    for c in (800, 400, 200, 100, 40, 8):
        if n % c == 0:
            return c
    raise ValueError(f"no chunk size for {n}")


def _gather_embed(t1a, t1b, t2, i1a, i1b, i2, a):
    n1 = i1a.shape[0]
    n2 = i2.shape[0]
    c1 = _pick_chunk(n1)
    c2 = _pick_chunk(n2)
    g1 = n1 // c1
    g2 = n2 // c2
    k1 = -(-g1 // _NW)
    k2 = -(-g2 // _NW)
    nvec = a // _LANES

    mesh = plsc.VectorSubcoreMesh(
        core_axis_name="c", subcore_axis_name="s", num_cores=_NC, num_subcores=_NS
    )

    @functools.partial(
        pl.kernel,
        out_type=jax.ShapeDtypeStruct((n1 + n2, a), jnp.float32),
        mesh=mesh,
        compiler_params=pltpu.CompilerParams(use_tc_tiling_on_sc=False),
        scratch_types=[
            pltpu.VMEM((c1,), jnp.int32),
            pltpu.VMEM((c1,), jnp.int32),
            pltpu.VMEM((c1, a), jnp.float32),
            pltpu.VMEM((c1, a), jnp.float32),
            pltpu.SemaphoreType.DMA,
        ],
    )
    def sc_kernel(t1a_h, t1b_h, t2_h, i1a_h, i1b_h, i2_h, out_h, ia_v, ib_v, ra_v, rb_v, sem):
        wid = lax.axis_index("s") * _NC + lax.axis_index("c")

        def p1_chunk(k, carry):
            g = wid + k * _NW

            @pl.when(g < g1)
            def _():
                base = g * c1
                pltpu.sync_copy(i1a_h.at[pl.ds(base, c1)], ia_v)
                pltpu.sync_copy(i1b_h.at[pl.ds(base, c1)], ib_v)
                da = pltpu.async_copy(t1a_h.at[ia_v], ra_v, sem)
                db = pltpu.async_copy(t1b_h.at[ib_v], rb_v, sem)
                da.wait()
                db.wait()

                def add_row(r, carry2):
                    for j in range(nvec):
                        sl = pl.ds(j * _LANES, _LANES)
                        plsc.addupdate(ra_v.at[r, sl], rb_v[r, sl])
                    return carry2

                lax.fori_loop(0, c1, add_row, 0)
                pltpu.sync_copy(ra_v, out_h.at[pl.ds(base, c1)])

            return carry

        lax.fori_loop(0, k1, p1_chunk, 0)

        def p2_chunk(k, carry):
            g = wid + k * _NW

            @pl.when(g < g2)
            def _():
                base = g * c2
                pltpu.sync_copy(i2_h.at[pl.ds(base, c2)], ia_v)
                pltpu.async_copy(t2_h.at[ia_v], ra_v, sem).wait()
                pltpu.sync_copy(ra_v, out_h.at[pl.ds(n1 + base, c2)])

            return carry

        lax.fori_loop(0, k2, p2_chunk, 0)

    return sc_kernel(t1a, t1b, t2, i1a, i1b, i2)


def kernel(constants_entity, tuples_p1, tuples_p2, W_p1, b_p1, W_p2, b_p2):
    t1a, t1b, t2 = _make_tables(constants_entity, W_p1, b_p1, W_p2, b_p2)
    i1a = tuples_p1[:, 0].astype(jnp.int32)
    i1b = tuples_p1[:, 1].astype(jnp.int32)
    i2 = tuples_p2[:, 0].astype(jnp.int32)
    return _gather_embed(t1a, t1b, t2, i1a, i1b, i2, W_p1.shape[1])
